# baseline (speedup 1.0000x reference)
import functools

import jax
import jax.numpy as jnp
from jax import lax
from jax.experimental import pallas as pl
from jax.experimental.pallas import tpu as pltpu
from jax.experimental.pallas import tpu_sc as plsc

_N = 1_000_000
_D = 64
_Q = 64
_NP = _N // 2
_BLKP = 5_000
_EPS = 1e-12


def _norm_rows(x):
    return x / jnp.maximum(
        jnp.sqrt(jnp.sum(x * x, axis=1, keepdims=True)), _EPS)


_CHUNK = _BLKP * 2 * _D


def _topk_body(q_ref, m_hbm, conf_ref, idx_ref, mbuf, sems):
    i = pl.program_id(0)
    n = pl.num_programs(0)
    slot = lax.rem(i, 2)

    @pl.when(i == 0)
    def _init():
        conf_ref[...] = jnp.full((1, _Q), -jnp.inf, jnp.float32)
        idx_ref[...] = jnp.zeros((1, _Q), jnp.int32)
        pltpu.make_async_copy(
            m_hbm.at[pl.ds(0, _CHUNK)], mbuf.at[0], sems.at[0]).start()

    @pl.when(i + 1 < n)
    def _prefetch():
        nxt = lax.rem(i + 1, 2)
        pltpu.make_async_copy(
            m_hbm.at[pl.ds((i + 1) * _CHUNK, _CHUNK)], mbuf.at[nxt],
            sems.at[nxt]).start()

    pltpu.make_async_copy(
        m_hbm.at[pl.ds(i * _CHUNK, _CHUNK)], mbuf.at[slot],
        sems.at[slot]).wait()

    qn = _norm_rows(q_ref[...])
    mp = jnp.reshape(mbuf[slot], (_BLKP, 2 * _D))
    mn_e = _norm_rows(mp[:, :_D])
    mn_o = _norm_rows(mp[:, _D:])
    sims_e = lax.dot_general(qn, mn_e, (((1,), (1,)), ((), ())),
                             preferred_element_type=jnp.float32)
    sims_o = lax.dot_general(qn, mn_o, (((1,), (1,)), ((), ())),
                             preferred_element_type=jnp.float32)

    col = lax.broadcasted_iota(jnp.int32, sims_e.shape, 1)
    vmax_e = jnp.max(sims_e, axis=1)
    arg_e = jnp.min(jnp.where(sims_e == vmax_e[:, None], col, _BLKP), axis=1)
    vmax_o = jnp.max(sims_o, axis=1)
    arg_o = jnp.min(jnp.where(sims_o == vmax_o[:, None], col, _BLKP), axis=1)

    base = i * _BLKP
    ge = 2 * (base + arg_e)
    go = 2 * (base + arg_o) + 1
    take_o = (vmax_o > vmax_e) | ((vmax_o == vmax_e) & (go < ge))
    v = jnp.where(take_o, vmax_o, vmax_e)
    g = jnp.where(take_o, go, ge)

    run_v = conf_ref[0, :]
    upd = v > run_v
    conf_ref[0, :] = jnp.where(upd, v, run_v)
    idx_ref[0, :] = jnp.where(upd, g, idx_ref[0, :])


_topk_call = pl.pallas_call(
    _topk_body,
    grid=(_NP // _BLKP,),
    in_specs=[
        pl.BlockSpec((_Q, _D), lambda i: (0, 0)),
        pl.BlockSpec(memory_space=pl.ANY),
    ],
    out_specs=[
        pl.BlockSpec((1, _Q), lambda i: (0, 0)),
        pl.BlockSpec((1, _Q), lambda i: (0, 0)),
    ],
    out_shape=[
        jax.ShapeDtypeStruct((1, _Q), jnp.float32),
        jax.ShapeDtypeStruct((1, _Q), jnp.int32),
    ],
    scratch_shapes=[
        pltpu.VMEM((2, _CHUNK), jnp.float32),
        pltpu.SemaphoreType.DMA((2,)),
    ],
)


def _sc_gather_body(values_hbm, idx_hbm, out_hbm, idx_v, rows_v, sem):
    wid = lax.axis_index("s") * 2 + lax.axis_index("c")

    @pl.when(wid == 0)
    def _():
        pltpu.sync_copy(idx_hbm, idx_v)
        pltpu.async_copy(values_hbm.at[idx_v], rows_v, sem).wait()
        pltpu.sync_copy(rows_v, out_hbm)


_sc_gather = functools.partial(
    pl.kernel,
    out_type=jax.ShapeDtypeStruct((_Q,), jnp.int32),
    mesh=plsc.VectorSubcoreMesh(core_axis_name="c", subcore_axis_name="s"),
    scratch_types=[
        pltpu.VMEM((_Q,), jnp.int32),
        pltpu.VMEM((_Q,), jnp.int32),
        pltpu.SemaphoreType.DMA,
    ],
)(_sc_gather_body)


def kernel(query_features, memory_keys, memory_values):
    mk_flat = memory_keys.reshape(_N * _D)
    conf2, idx2 = _topk_call(query_features, mk_flat)
    confidence = conf2[0]
    indices = idx2[0]
    retrieved = _sc_gather(memory_values, indices)
    return retrieved, confidence


# speedup vs baseline: 1.0028x; 1.0028x over previous
import functools

import jax
import jax.numpy as jnp
from jax import lax
from jax.experimental import pallas as pl
from jax.experimental.pallas import tpu as pltpu
from jax.experimental.pallas import tpu_sc as plsc

_N = 1_000_000
_D = 64
_Q = 64
_NP = _N // 2
_BLKP = 5_000
_EPS = 1e-12


def _norm_rows(x):
    return x / jnp.maximum(
        jnp.sqrt(jnp.sum(x * x, axis=1, keepdims=True)), _EPS)


_CHUNK = _BLKP * 2 * _D


def _topk_body(q_ref, m_ref, conf_ref, idx_ref):
    i = pl.program_id(0)

    @pl.when(i == 0)
    def _init():
        conf_ref[...] = jnp.full((1, _Q), -jnp.inf, jnp.float32)
        idx_ref[...] = jnp.zeros((1, _Q), jnp.int32)

    qn = _norm_rows(q_ref[...])
    mp = m_ref[...]
    mn_e = _norm_rows(mp[:, :_D])
    mn_o = _norm_rows(mp[:, _D:])
    sims_e = lax.dot_general(qn, mn_e, (((1,), (1,)), ((), ())),
                             preferred_element_type=jnp.float32)
    sims_o = lax.dot_general(qn, mn_o, (((1,), (1,)), ((), ())),
                             preferred_element_type=jnp.float32)

    col = lax.broadcasted_iota(jnp.int32, sims_e.shape, 1)
    vmax_e = jnp.max(sims_e, axis=1)
    arg_e = jnp.min(jnp.where(sims_e == vmax_e[:, None], col, _BLKP), axis=1)
    vmax_o = jnp.max(sims_o, axis=1)
    arg_o = jnp.min(jnp.where(sims_o == vmax_o[:, None], col, _BLKP), axis=1)

    base = i * _BLKP
    ge = 2 * (base + arg_e)
    go = 2 * (base + arg_o) + 1
    take_o = (vmax_o > vmax_e) | ((vmax_o == vmax_e) & (go < ge))
    v = jnp.where(take_o, vmax_o, vmax_e)
    g = jnp.where(take_o, go, ge)

    run_v = conf_ref[0, :]
    upd = v > run_v
    conf_ref[0, :] = jnp.where(upd, v, run_v)
    idx_ref[0, :] = jnp.where(upd, g, idx_ref[0, :])


_topk_call = pl.pallas_call(
    _topk_body,
    grid=(_NP // _BLKP,),
    in_specs=[
        pl.BlockSpec((_Q, _D), lambda i: (0, 0)),
        pl.BlockSpec((_BLKP, 2 * _D), lambda i: (i, 0)),
    ],
    out_specs=[
        pl.BlockSpec((1, _Q), lambda i: (0, 0)),
        pl.BlockSpec((1, _Q), lambda i: (0, 0)),
    ],
    out_shape=[
        jax.ShapeDtypeStruct((1, _Q), jnp.float32),
        jax.ShapeDtypeStruct((1, _Q), jnp.int32),
    ],
    compiler_params=pltpu.CompilerParams(
        allow_input_fusion=[False, True],
    ),
)


def _sc_gather_body(values_hbm, idx_hbm, out_hbm, idx_v, rows_v, sem):
    wid = lax.axis_index("s") * 2 + lax.axis_index("c")

    @pl.when(wid == 0)
    def _():
        pltpu.sync_copy(idx_hbm, idx_v)
        pltpu.async_copy(values_hbm.at[idx_v], rows_v, sem).wait()
        pltpu.sync_copy(rows_v, out_hbm)


_sc_gather = functools.partial(
    pl.kernel,
    out_type=jax.ShapeDtypeStruct((_Q,), jnp.int32),
    mesh=plsc.VectorSubcoreMesh(core_axis_name="c", subcore_axis_name="s"),
    scratch_types=[
        pltpu.VMEM((_Q,), jnp.int32),
        pltpu.VMEM((_Q,), jnp.int32),
        pltpu.SemaphoreType.DMA,
    ],
)(_sc_gather_body)


def kernel(query_features, memory_keys, memory_values):
    mp = memory_keys.reshape(_NP, 2 * _D)
    conf2, idx2 = _topk_call(query_features, mp)
    confidence = conf2[0]
    indices = idx2[0]
    retrieved = _sc_gather(memory_values, indices)
    return retrieved, confidence


# speedup vs baseline: 6.4900x; 6.4721x over previous
import functools

import jax
import jax.numpy as jnp
from jax import lax
from jax.experimental import pallas as pl
from jax.experimental.pallas import tpu as pltpu
from jax.experimental.pallas import tpu_sc as plsc

_N = 1_000_000
_D = 64
_Q = 64
_C = 16_384
_GRID = -(-_N // _C)
_LAST_VALID = _N - (_GRID - 1) * _C
_EPS = 1e-12


def _topk_body(q_ref, mt_ref, conf_ref, idx_ref):
    i = pl.program_id(0)

    @pl.when(i == 0)
    def _init():
        conf_ref[...] = jnp.full((1, _Q), -jnp.inf, jnp.float32)
        idx_ref[...] = jnp.zeros((1, _Q), jnp.int32)

    q = q_ref[...]
    qn = q / jnp.maximum(
        jnp.sqrt(jnp.sum(q * q, axis=1, keepdims=True)), _EPS)
    mt = mt_ref[...]
    n = jnp.maximum(jnp.sqrt(jnp.sum(mt * mt, axis=0, keepdims=True)), _EPS)
    mn = mt / n
    sims = lax.dot_general(qn, mn, (((1,), (0,)), ((), ())),
                           preferred_element_type=jnp.float32)

    col = lax.broadcasted_iota(jnp.int32, sims.shape, 1)
    sims = jnp.where((i == _GRID - 1) & (col >= _LAST_VALID),
                     -jnp.inf, sims)

    local_max = jnp.max(sims, axis=1)
    local_arg = jnp.min(
        jnp.where(sims == local_max[:, None], col, _C), axis=1)

    run_v = conf_ref[0, :]
    upd = local_max > run_v
    conf_ref[0, :] = jnp.where(upd, local_max, run_v)
    idx_ref[0, :] = jnp.where(upd, i * _C + local_arg, idx_ref[0, :])


_topk_call = pl.pallas_call(
    _topk_body,
    grid=(_GRID,),
    in_specs=[
        pl.BlockSpec((_Q, _D), lambda i: (0, 0)),
        pl.BlockSpec((_D, _C), lambda i: (0, i)),
    ],
    out_specs=[
        pl.BlockSpec((1, _Q), lambda i: (0, 0)),
        pl.BlockSpec((1, _Q), lambda i: (0, 0)),
    ],
    out_shape=[
        jax.ShapeDtypeStruct((1, _Q), jnp.float32),
        jax.ShapeDtypeStruct((1, _Q), jnp.int32),
    ],
)


def _sc_gather_body(values_hbm, idx_hbm, out_hbm, idx_v, rows_v, sem):
    wid = lax.axis_index("s") * 2 + lax.axis_index("c")

    @pl.when(wid == 0)
    def _():
        pltpu.sync_copy(idx_hbm, idx_v)
        pltpu.async_copy(values_hbm.at[idx_v], rows_v, sem).wait()
        pltpu.sync_copy(rows_v, out_hbm)


_sc_gather = functools.partial(
    pl.kernel,
    out_type=jax.ShapeDtypeStruct((_Q,), jnp.int32),
    mesh=plsc.VectorSubcoreMesh(core_axis_name="c", subcore_axis_name="s"),
    scratch_types=[
        pltpu.VMEM((_Q,), jnp.int32),
        pltpu.VMEM((_Q,), jnp.int32),
        pltpu.SemaphoreType.DMA,
    ],
)(_sc_gather_body)


def kernel(query_features, memory_keys, memory_values):
    mt = memory_keys.T
    conf2, idx2 = _topk_call(query_features, mt)
    confidence = conf2[0]
    indices = idx2[0]
    retrieved = _sc_gather(memory_values, indices)
    return retrieved, confidence


# speedup vs baseline: 6.5616x; 1.0110x over previous
import functools

import jax
import jax.numpy as jnp
from jax import lax
from jax.experimental import pallas as pl
from jax.experimental.pallas import tpu as pltpu
from jax.experimental.pallas import tpu_sc as plsc

_N = 1_000_000
_D = 64
_Q = 64
_C = 16_384
_GRID = -(-_N // _C)
_LAST_VALID = _N - (_GRID - 1) * _C
_EPS = 1e-12


def _topk_body(q_ref, mt_ref, conf_ref, idx_ref):
    i = pl.program_id(0)

    @pl.when(i == 0)
    def _init():
        conf_ref[...] = jnp.full((1, _Q), -jnp.inf, jnp.float32)
        idx_ref[...] = jnp.zeros((1, _Q), jnp.int32)

    q = q_ref[...]
    qn = q / jnp.maximum(
        jnp.sqrt(jnp.sum(q * q, axis=1, keepdims=True)), _EPS)
    mt = mt_ref[...]
    n = jnp.maximum(jnp.sqrt(jnp.sum(mt * mt, axis=0, keepdims=True)), _EPS)
    mn = mt / n
    sims = lax.dot_general(qn, mn, (((1,), (0,)), ((), ())),
                           preferred_element_type=jnp.float32)

    colf = lax.broadcasted_iota(jnp.int32, (1, _C), 1).astype(jnp.float32)

    thresh = jnp.where(i == _GRID - 1, float(_LAST_VALID), jnp.inf)
    sims = jnp.where(colf >= thresh, -jnp.inf, sims)

    local_max = jnp.max(sims, axis=1)
    local_arg = jnp.min(
        jnp.where(sims == local_max[:, None], colf, float(_C)),
        axis=1).astype(jnp.int32)

    run_v = conf_ref[0, :]
    upd = local_max > run_v
    conf_ref[0, :] = jnp.where(upd, local_max, run_v)
    idx_ref[0, :] = jnp.where(upd, i * _C + local_arg, idx_ref[0, :])


_topk_call = pl.pallas_call(
    _topk_body,
    grid=(_GRID,),
    in_specs=[
        pl.BlockSpec((_Q, _D), lambda i: (0, 0)),
        pl.BlockSpec((_D, _C), lambda i: (0, i)),
    ],
    out_specs=[
        pl.BlockSpec((1, _Q), lambda i: (0, 0)),
        pl.BlockSpec((1, _Q), lambda i: (0, 0)),
    ],
    out_shape=[
        jax.ShapeDtypeStruct((1, _Q), jnp.float32),
        jax.ShapeDtypeStruct((1, _Q), jnp.int32),
    ],
)


def _sc_gather_body(values_hbm, idx_hbm, out_hbm, idx_v, rows_v, sem):
    wid = lax.axis_index("s") * 2 + lax.axis_index("c")

    @pl.when(wid == 0)
    def _():
        pltpu.sync_copy(idx_hbm, idx_v)
        pltpu.async_copy(values_hbm.at[idx_v], rows_v, sem).wait()
        pltpu.sync_copy(rows_v, out_hbm)


_sc_gather = functools.partial(
    pl.kernel,
    out_type=jax.ShapeDtypeStruct((_Q,), jnp.int32),
    mesh=plsc.VectorSubcoreMesh(core_axis_name="c", subcore_axis_name="s"),
    scratch_types=[
        pltpu.VMEM((_Q,), jnp.int32),
        pltpu.VMEM((_Q,), jnp.int32),
        pltpu.SemaphoreType.DMA,
    ],
)(_sc_gather_body)


def kernel(query_features, memory_keys, memory_values):
    mt = memory_keys.T
    conf2, idx2 = _topk_call(query_features, mt)
    confidence = conf2[0]
    indices = idx2[0]
    retrieved = _sc_gather(memory_values, indices)
    return retrieved, confidence


# speedup vs baseline: 7.1362x; 1.0876x over previous
import functools

import jax
import jax.numpy as jnp
from jax import lax
from jax.experimental import pallas as pl
from jax.experimental.pallas import tpu as pltpu
from jax.experimental.pallas import tpu_sc as plsc

_N = 1_000_000
_D = 64
_Q = 64
_C = 32_768
_GRID = -(-_N // _C)
_LAST_VALID = _N - (_GRID - 1) * _C
_EPS = 1e-12


def _topk_body(q_ref, mt_ref, conf_ref, idx_ref):
    i = pl.program_id(0)

    @pl.when(i == 0)
    def _init():
        conf_ref[...] = jnp.full((1, _Q), -jnp.inf, jnp.float32)
        idx_ref[...] = jnp.zeros((1, _Q), jnp.int32)

    q = q_ref[...]
    qn = q / jnp.maximum(
        jnp.sqrt(jnp.sum(q * q, axis=1, keepdims=True)), _EPS)
    mt = mt_ref[...]
    n = jnp.maximum(jnp.sqrt(jnp.sum(mt * mt, axis=0, keepdims=True)), _EPS)
    mn = mt / n
    sims = lax.dot_general(qn, mn, (((1,), (0,)), ((), ())),
                           preferred_element_type=jnp.float32)

    colf = lax.broadcasted_iota(jnp.int32, (1, _C), 1).astype(jnp.float32)

    thresh = jnp.where(i == _GRID - 1, float(_LAST_VALID), jnp.inf)
    sims = jnp.where(colf >= thresh, -jnp.inf, sims)

    local_max = jnp.max(sims, axis=1)
    local_arg = jnp.min(
        jnp.where(sims == local_max[:, None], colf, float(_C)),
        axis=1).astype(jnp.int32)

    run_v = conf_ref[0, :]
    upd = local_max > run_v
    conf_ref[0, :] = jnp.where(upd, local_max, run_v)
    idx_ref[0, :] = jnp.where(upd, i * _C + local_arg, idx_ref[0, :])


_topk_call = pl.pallas_call(
    _topk_body,
    grid=(_GRID,),
    in_specs=[
        pl.BlockSpec((_Q, _D), lambda i: (0, 0)),
        pl.BlockSpec((_D, _C), lambda i: (0, i)),
    ],
    out_specs=[
        pl.BlockSpec((1, _Q), lambda i: (0, 0)),
        pl.BlockSpec((1, _Q), lambda i: (0, 0)),
    ],
    out_shape=[
        jax.ShapeDtypeStruct((1, _Q), jnp.float32),
        jax.ShapeDtypeStruct((1, _Q), jnp.int32),
    ],
)


def _sc_gather_body(values_hbm, idx_hbm, out_hbm, idx_v, rows_v, sem):
    wid = lax.axis_index("s") * 2 + lax.axis_index("c")

    @pl.when(wid == 0)
    def _():
        pltpu.sync_copy(idx_hbm, idx_v)
        pltpu.async_copy(values_hbm.at[idx_v], rows_v, sem).wait()
        pltpu.sync_copy(rows_v, out_hbm)


_sc_gather = functools.partial(
    pl.kernel,
    out_type=jax.ShapeDtypeStruct((_Q,), jnp.int32),
    mesh=plsc.VectorSubcoreMesh(core_axis_name="c", subcore_axis_name="s"),
    scratch_types=[
        pltpu.VMEM((_Q,), jnp.int32),
        pltpu.VMEM((_Q,), jnp.int32),
        pltpu.SemaphoreType.DMA,
    ],
)(_sc_gather_body)


def kernel(query_features, memory_keys, memory_values):
    mt = memory_keys.T
    conf2, idx2 = _topk_call(query_features, mt)
    confidence = conf2[0]
    indices = idx2[0]
    retrieved = _sc_gather(memory_values, indices)
    return retrieved, confidence


# speedup vs baseline: 7.9231x; 1.1103x over previous
import functools

import jax
import jax.numpy as jnp
from jax import lax
from jax.experimental import pallas as pl
from jax.experimental.pallas import tpu as pltpu
from jax.experimental.pallas import tpu_sc as plsc

_N = 1_000_000
_D = 64
_Q = 64
_C = 32_768
_GRID = -(-_N // _C)
_LAST_VALID = _N - (_GRID - 1) * _C
_EPS = 1e-12


def _topk_body(q_ref, mt_ref, conf_ref, idx_ref):
    i = pl.program_id(0)

    @pl.when(i == 0)
    def _init():
        conf_ref[...] = jnp.full((1, _Q), -jnp.inf, jnp.float32)
        idx_ref[...] = jnp.zeros((1, _Q), jnp.int32)

    q = q_ref[...]
    qn = q / jnp.maximum(
        jnp.sqrt(jnp.sum(q * q, axis=1, keepdims=True)), _EPS)
    mt = mt_ref[...]
    colf = lax.broadcasted_iota(jnp.int32, (1, _C), 1).astype(jnp.float32)

    n = jnp.maximum(jnp.sqrt(jnp.sum(mt * mt, axis=0, keepdims=True)), _EPS)
    thresh = jnp.where(i == _GRID - 1, float(_LAST_VALID), jnp.inf)
    n = jnp.where(colf >= thresh, jnp.inf, n)
    mn = mt * (1.0 / n)
    sims = lax.dot_general(qn, mn, (((1,), (0,)), ((), ())),
                           preferred_element_type=jnp.float32)

    local_max = jnp.max(sims, axis=1)
    local_arg = jnp.min(
        jnp.where(sims == local_max[:, None], colf, float(_C)),
        axis=1).astype(jnp.int32)

    run_v = conf_ref[0, :]
    upd = local_max > run_v
    conf_ref[0, :] = jnp.where(upd, local_max, run_v)
    idx_ref[0, :] = jnp.where(upd, i * _C + local_arg, idx_ref[0, :])


_topk_call = pl.pallas_call(
    _topk_body,
    grid=(_GRID,),
    in_specs=[
        pl.BlockSpec((_Q, _D), lambda i: (0, 0)),
        pl.BlockSpec((_D, _C), lambda i: (0, i)),
    ],
    out_specs=[
        pl.BlockSpec((1, _Q), lambda i: (0, 0)),
        pl.BlockSpec((1, _Q), lambda i: (0, 0)),
    ],
    out_shape=[
        jax.ShapeDtypeStruct((1, _Q), jnp.float32),
        jax.ShapeDtypeStruct((1, _Q), jnp.int32),
    ],
)


def _sc_gather_body(values_hbm, idx_hbm, out_hbm, idx_v, rows_v, sem):
    wid = lax.axis_index("s") * 2 + lax.axis_index("c")

    @pl.when(wid == 0)
    def _():
        pltpu.sync_copy(idx_hbm, idx_v)
        pltpu.async_copy(values_hbm.at[idx_v], rows_v, sem).wait()
        pltpu.sync_copy(rows_v, out_hbm)


_sc_gather = functools.partial(
    pl.kernel,
    out_type=jax.ShapeDtypeStruct((_Q,), jnp.int32),
    mesh=plsc.VectorSubcoreMesh(core_axis_name="c", subcore_axis_name="s"),
    scratch_types=[
        pltpu.VMEM((_Q,), jnp.int32),
        pltpu.VMEM((_Q,), jnp.int32),
        pltpu.SemaphoreType.DMA,
    ],
)(_sc_gather_body)


def kernel(query_features, memory_keys, memory_values):
    mt = memory_keys.T
    conf2, idx2 = _topk_call(query_features, mt)
    confidence = conf2[0]
    indices = idx2[0]
    retrieved = _sc_gather(memory_values, indices)
    return retrieved, confidence


# speedup vs baseline: 7.9598x; 1.0046x over previous
import functools

import jax
import jax.numpy as jnp
from jax import lax
from jax.experimental import pallas as pl
from jax.experimental.pallas import tpu as pltpu
from jax.experimental.pallas import tpu_sc as plsc

_N = 1_000_000
_D = 64
_Q = 64
_C = 65_536
_GRID = -(-_N // _C)
_LAST_VALID = _N - (_GRID - 1) * _C
_EPS = 1e-12


def _topk_body(q_ref, mt_ref, conf_ref, idx_ref):
    i = pl.program_id(0)

    @pl.when(i == 0)
    def _init():
        conf_ref[...] = jnp.full((1, _Q), -jnp.inf, jnp.float32)
        idx_ref[...] = jnp.zeros((1, _Q), jnp.int32)

    q = q_ref[...]
    qn = q / jnp.maximum(
        jnp.sqrt(jnp.sum(q * q, axis=1, keepdims=True)), _EPS)
    mt = mt_ref[...]
    colf = lax.broadcasted_iota(jnp.int32, (1, _C), 1).astype(jnp.float32)

    n = jnp.maximum(jnp.sqrt(jnp.sum(mt * mt, axis=0, keepdims=True)), _EPS)
    thresh = jnp.where(i == _GRID - 1, float(_LAST_VALID), jnp.inf)
    n = jnp.where(colf >= thresh, jnp.inf, n)
    mn = mt * (1.0 / n)
    sims = lax.dot_general(qn, mn, (((1,), (0,)), ((), ())),
                           preferred_element_type=jnp.float32)

    local_max = jnp.max(sims, axis=1)
    local_arg = jnp.min(
        jnp.where(sims == local_max[:, None], colf, float(_C)),
        axis=1).astype(jnp.int32)

    run_v = conf_ref[0, :]
    upd = local_max > run_v
    conf_ref[0, :] = jnp.where(upd, local_max, run_v)
    idx_ref[0, :] = jnp.where(upd, i * _C + local_arg, idx_ref[0, :])


_topk_call = pl.pallas_call(
    _topk_body,
    grid=(_GRID,),
    in_specs=[
        pl.BlockSpec((_Q, _D), lambda i: (0, 0)),
        pl.BlockSpec((_D, _C), lambda i: (0, i)),
    ],
    out_specs=[
        pl.BlockSpec((1, _Q), lambda i: (0, 0)),
        pl.BlockSpec((1, _Q), lambda i: (0, 0)),
    ],
    out_shape=[
        jax.ShapeDtypeStruct((1, _Q), jnp.float32),
        jax.ShapeDtypeStruct((1, _Q), jnp.int32),
    ],
)


def _sc_gather_body(values_hbm, idx_hbm, out_hbm, idx_v, rows_v, sem):
    wid = lax.axis_index("s") * 2 + lax.axis_index("c")

    @pl.when(wid == 0)
    def _():
        pltpu.sync_copy(idx_hbm, idx_v)
        pltpu.async_copy(values_hbm.at[idx_v], rows_v, sem).wait()
        pltpu.sync_copy(rows_v, out_hbm)


_sc_gather = functools.partial(
    pl.kernel,
    out_type=jax.ShapeDtypeStruct((_Q,), jnp.int32),
    mesh=plsc.VectorSubcoreMesh(core_axis_name="c", subcore_axis_name="s"),
    scratch_types=[
        pltpu.VMEM((_Q,), jnp.int32),
        pltpu.VMEM((_Q,), jnp.int32),
        pltpu.SemaphoreType.DMA,
    ],
)(_sc_gather_body)


def kernel(query_features, memory_keys, memory_values):
    mt = memory_keys.T
    conf2, idx2 = _topk_call(query_features, mt)
    confidence = conf2[0]
    indices = idx2[0]
    retrieved = _sc_gather(memory_values, indices)
    return retrieved, confidence
